# baseline (speedup 1.0000x reference)
import functools

import jax
import jax.numpy as jnp
from jax import lax
from jax.experimental import pallas as pl
from jax.experimental.pallas import tpu as pltpu
from jax.experimental.pallas import tpu_sc as plsc

N = 10000
E = 320000
HID = 128
CP = 16
NC = 2
NS = 16
NW = NC * NS
EW = E // NW
C = 80
K = EW // C
NP = 10112
TPR = NP // NS
BN = 2000
BE = 1280


def _silu(x):
    return x * jax.nn.sigmoid(x)


def _linear_body(x_ref, wt_ref, b_ref, o_ref):
    o_ref[...] = jnp.dot(x_ref[...], wt_ref[...],
                         preferred_element_type=jnp.float32) + b_ref[...]


def _linear(x, wt, b):
    return pl.pallas_call(
        _linear_body,
        grid=(N // BN,),
        in_specs=[pl.BlockSpec((BN, x.shape[1]), lambda i: (i, 0)),
                  pl.BlockSpec(wt.shape, lambda i: (0, 0)),
                  pl.BlockSpec((1, wt.shape[1]), lambda i: (0, 0))],
        out_specs=pl.BlockSpec((BN, wt.shape[1]), lambda i: (i, 0)),
        out_shape=jax.ShapeDtypeStruct((N, wt.shape[1]), jnp.float32),
    )(x, wt, b)


def _prep_body(x_ref, wct_ref, wrt_ref, b1_ref, p_ref, q_ref):
    x = x_ref[...]
    p_ref[...] = jnp.dot(x, wct_ref[...],
                         preferred_element_type=jnp.float32) + b1_ref[...]
    q_ref[...] = jnp.dot(x, wrt_ref[...], preferred_element_type=jnp.float32)


def _prep(x, wct, wrt, b1):
    return pl.pallas_call(
        _prep_body,
        grid=(N // BN,),
        in_specs=[pl.BlockSpec((BN, HID), lambda i: (i, 0)),
                  pl.BlockSpec((HID, HID), lambda i: (0, 0)),
                  pl.BlockSpec((HID, HID), lambda i: (0, 0)),
                  pl.BlockSpec((1, HID), lambda i: (0, 0))],
        out_specs=[pl.BlockSpec((BN, HID), lambda i: (i, 0)),
                   pl.BlockSpec((BN, HID), lambda i: (i, 0))],
        out_shape=[jax.ShapeDtypeStruct((N, HID), jnp.float32),
                   jax.ShapeDtypeStruct((N, HID), jnp.float32)],
    )(x, wct, wrt, b1)


def _edge_body(gf_ref, gc_ref, ea_ref, wd_ref, wat_ref,
               ew2t_ref, eb2_ref, cw1t_ref, cb1_ref, cw2_ref, m1_ref, m2_ref):
    cd3 = gc_ref[...][:, :3]
    dist = jnp.sum(cd3 * cd3, axis=1, keepdims=True)
    pre = (gf_ref[...] + dist * wd_ref[...]
           + jnp.dot(ea_ref[...], wat_ref[...],
                     preferred_element_type=jnp.float32))
    u = _silu(pre)
    ef = _silu(jnp.dot(u, ew2t_ref[...],
                       preferred_element_type=jnp.float32) + eb2_ref[...])
    t = _silu(jnp.dot(ef, cw1t_ref[...],
                      preferred_element_type=jnp.float32) + cb1_ref[...])
    cw = jnp.sum(t * cw2_ref[...], axis=1, keepdims=True)
    ones = jnp.ones((BE, 1), jnp.float32)
    zeros = jnp.zeros((BE, CP - 4), jnp.float32)
    m1_ref[...] = ef
    m2_ref[...] = jnp.concatenate([cd3 * cw, ones, zeros], axis=1)


def _edge(gf, gc, ea, wd, wat, ew2t, eb2, cw1t, cb1, cw2):
    return pl.pallas_call(
        _edge_body,
        grid=(E // BE,),
        in_specs=[pl.BlockSpec((BE, HID), lambda i: (i, 0)),
                  pl.BlockSpec((BE, CP), lambda i: (i, 0)),
                  pl.BlockSpec((BE, 4), lambda i: (i, 0)),
                  pl.BlockSpec((1, HID), lambda i: (0, 0)),
                  pl.BlockSpec((4, HID), lambda i: (0, 0)),
                  pl.BlockSpec((HID, HID), lambda i: (0, 0)),
                  pl.BlockSpec((1, HID), lambda i: (0, 0)),
                  pl.BlockSpec((HID, HID), lambda i: (0, 0)),
                  pl.BlockSpec((1, HID), lambda i: (0, 0)),
                  pl.BlockSpec((1, HID), lambda i: (0, 0))],
        out_specs=[pl.BlockSpec((BE, HID), lambda i: (i, 0)),
                   pl.BlockSpec((BE, CP), lambda i: (i, 0))],
        out_shape=[jax.ShapeDtypeStruct((E, HID), jnp.float32),
                   jax.ShapeDtypeStruct((E, CP), jnp.float32)],
    )(gf, gc, ea, wd, wat, ew2t, eb2, cw1t, cb1, cw2)


def _node_body(x_ref, cp_ref, s1a_ref, s1b_ref, s2a_ref, s2b_ref,
               w1at_ref, w1bt_ref, b1_ref, w2t_ref, b2_ref,
               xo_ref, cpo_ref, cno_ref):
    x = x_ref[...]
    agg = s1a_ref[0] + s1b_ref[0]
    t2 = s2a_ref[0] + s2b_ref[0]
    csum = t2[:, :3]
    cnt = t2[:, 3:4]
    aggc = csum / jnp.maximum(cnt, 1.0)
    pre = (jnp.dot(x, w1at_ref[...], preferred_element_type=jnp.float32)
           + jnp.dot(agg, w1bt_ref[...], preferred_element_type=jnp.float32)
           + b1_ref[...])
    upd = jnp.dot(_silu(pre), w2t_ref[...],
                  preferred_element_type=jnp.float32) + b2_ref[...]
    xo_ref[...] = x + upd
    cpo = cp_ref[...] + jnp.concatenate(
        [aggc, jnp.zeros((BN, CP - 3), jnp.float32)], axis=1)
    cpo_ref[...] = cpo
    cno_ref[...] = -cpo


def _node(x, cpad, s1, s2, w1at, w1bt, b1, w2t, b2):
    return pl.pallas_call(
        _node_body,
        grid=(N // BN,),
        in_specs=[pl.BlockSpec((BN, HID), lambda i: (i, 0)),
                  pl.BlockSpec((BN, CP), lambda i: (i, 0)),
                  pl.BlockSpec((1, BN, HID), lambda i: (0, i, 0)),
                  pl.BlockSpec((1, BN, HID), lambda i: (1, i, 0)),
                  pl.BlockSpec((1, BN, CP), lambda i: (0, i, 0)),
                  pl.BlockSpec((1, BN, CP), lambda i: (1, i, 0)),
                  pl.BlockSpec((HID, HID), lambda i: (0, 0)),
                  pl.BlockSpec((HID, HID), lambda i: (0, 0)),
                  pl.BlockSpec((1, HID), lambda i: (0, 0)),
                  pl.BlockSpec((HID, HID), lambda i: (0, 0)),
                  pl.BlockSpec((1, HID), lambda i: (0, 0))],
        out_specs=[pl.BlockSpec((BN, HID), lambda i: (i, 0)),
                   pl.BlockSpec((BN, CP), lambda i: (i, 0)),
                   pl.BlockSpec((BN, CP), lambda i: (i, 0))],
        out_shape=[jax.ShapeDtypeStruct((N, HID), jnp.float32),
                   jax.ShapeDtypeStruct((N, CP), jnp.float32),
                   jax.ShapeDtypeStruct((N, CP), jnp.float32)],
    )(x, cpad, s1, s1, s2, s2, w1at, w1bt, b1, w2t, b2)


@functools.lru_cache(maxsize=1)
def _sc_mesh():
    return plsc.VectorSubcoreMesh(core_axis_name="c", subcore_axis_name="s")


def _add_bufs(buf_af, buf_bf, buf_ac, buf_bc):
    def add_row(r, carry):
        for l in range(HID // 16):
            sl = pl.ds(l * 16, 16)
            buf_af[r, sl] = buf_af[r, sl] + buf_bf[r, sl]
        buf_ac[r, :] = buf_ac[r, :] + buf_bc[r, :]
        return carry

    lax.fori_loop(0, C, add_row, 0)


def _gather_body(pf_hbm, qf_hbm, pc_hbm, qc_hbm, col_hbm, row_hbm,
                 gf_hbm, gc_hbm,
                 idxc, idxr,
                 buf_af0, buf_bf0, buf_ac0, buf_bc0,
                 buf_af1, buf_bf1, buf_ac1, buf_bc1,
                 sem0, sem1):
    wid = lax.axis_index("s") * NC + lax.axis_index("c")
    pltpu.sync_copy(col_hbm.at[wid], idxc)
    pltpu.sync_copy(row_hbm.at[wid], idxr)

    def fire(j, bf, bff, bc, bcc, sem):
        a = pltpu.async_copy(pf_hbm.at[idxc.at[j]], bf, sem)
        b = pltpu.async_copy(qf_hbm.at[idxr.at[j]], bff, sem)
        c = pltpu.async_copy(pc_hbm.at[idxc.at[j]], bc, sem)
        d = pltpu.async_copy(qc_hbm.at[idxr.at[j]], bcc, sem)
        return a, b, c, d

    def drain(descs):
        for t in descs:
            t.wait()

    def body(jj, carry):
        j0 = 2 * jj
        j1 = 2 * jj + 1
        d0 = fire(j0, buf_af0, buf_bf0, buf_ac0, buf_bc0, sem0)
        drain(d0)
        d1 = fire(j1, buf_af1, buf_bf1, buf_ac1, buf_bc1, sem1)
        _add_bufs(buf_af0, buf_bf0, buf_ac0, buf_bc0)
        base0 = wid * EW + j0 * C
        pltpu.sync_copy(buf_af0, gf_hbm.at[pl.ds(base0, C)])
        pltpu.sync_copy(buf_ac0, gc_hbm.at[pl.ds(base0, C)])
        drain(d1)
        _add_bufs(buf_af1, buf_bf1, buf_ac1, buf_bc1)
        base1 = wid * EW + j1 * C
        pltpu.sync_copy(buf_af1, gf_hbm.at[pl.ds(base1, C)])
        pltpu.sync_copy(buf_ac1, gc_hbm.at[pl.ds(base1, C)])
        return carry

    lax.fori_loop(0, K // 2, body, 0)
    if K % 2:
        j = K - 1
        drain(fire(j, buf_af0, buf_bf0, buf_ac0, buf_bc0, sem0))
        _add_bufs(buf_af0, buf_bf0, buf_ac0, buf_bc0)
        base = wid * EW + j * C
        pltpu.sync_copy(buf_af0, gf_hbm.at[pl.ds(base, C)])
        pltpu.sync_copy(buf_ac0, gc_hbm.at[pl.ds(base, C)])


def _sc_gather(pf, qf, pc, qc, col3, row3):
    kfn = pl.kernel(
        _gather_body,
        out_type=[jax.ShapeDtypeStruct((E, HID), jnp.float32),
                  jax.ShapeDtypeStruct((E, CP), jnp.float32)],
        mesh=_sc_mesh(),
        scratch_types=[pltpu.VMEM((K, C), jnp.int32),
                       pltpu.VMEM((K, C), jnp.int32),
                       pltpu.VMEM((C, HID), jnp.float32),
                       pltpu.VMEM((C, HID), jnp.float32),
                       pltpu.VMEM((C, CP), jnp.float32),
                       pltpu.VMEM((C, CP), jnp.float32),
                       pltpu.VMEM((C, HID), jnp.float32),
                       pltpu.VMEM((C, HID), jnp.float32),
                       pltpu.VMEM((C, CP), jnp.float32),
                       pltpu.VMEM((C, CP), jnp.float32),
                       pltpu.SemaphoreType.DMA,
                       pltpu.SemaphoreType.DMA],
        compiler_params=pltpu.CompilerParams(use_tc_tiling_on_sc=False),
    )
    return kfn(pf, qf, pc, qc, col3, row3)


def _scatter_body(m1_hbm, m2_hbm, col_hbm, z1_hbm, z2_hbm, s1_hbm, s2_hbm,
                  idxc, buf1, buf2, acc1, acc2):
    cid = lax.axis_index("c")
    sid = lax.axis_index("s")
    wid = sid * NC + cid
    pltpu.sync_copy(z1_hbm.at[pl.ds(sid * TPR, TPR)],
                    acc1.at[pl.ds(sid * TPR, TPR)])
    pltpu.sync_copy(z2_hbm.at[pl.ds(sid * TPR, TPR)],
                    acc2.at[pl.ds(sid * TPR, TPR)])
    pltpu.sync_copy(col_hbm.at[wid], idxc)
    plsc.subcore_barrier()

    def body(j, carry):
        base = wid * EW + j * C
        pltpu.sync_copy(m1_hbm.at[pl.ds(base, C)], buf1)
        pltpu.sync_copy(m2_hbm.at[pl.ds(base, C)], buf2)
        pltpu.sync_copy(buf1, acc1.at[idxc.at[j]], add=True)
        pltpu.sync_copy(buf2, acc2.at[idxc.at[j]], add=True)
        return carry

    lax.fori_loop(0, K, body, 0)
    plsc.subcore_barrier()
    pltpu.sync_copy(acc1.at[pl.ds(sid * TPR, TPR)],
                    s1_hbm.at[cid, pl.ds(sid * TPR, TPR)])
    pltpu.sync_copy(acc2.at[pl.ds(sid * TPR, TPR)],
                    s2_hbm.at[cid, pl.ds(sid * TPR, TPR)])


def _sc_scatter(m1, m2, col3, zeros1, zeros2):
    kfn = pl.kernel(
        _scatter_body,
        out_type=[jax.ShapeDtypeStruct((2, NP, HID), jnp.float32),
                  jax.ShapeDtypeStruct((2, NP, CP), jnp.float32)],
        mesh=_sc_mesh(),
        scratch_types=[pltpu.VMEM((K, C), jnp.int32),
                       pltpu.VMEM((C, HID), jnp.float32),
                       pltpu.VMEM((C, CP), jnp.float32),
                       pltpu.VMEM_SHARED((NP, HID), jnp.float32),
                       pltpu.VMEM_SHARED((NP, CP), jnp.float32)],
        compiler_params=pltpu.CompilerParams(use_tc_tiling_on_sc=False),
    )
    return kfn(m1, m2, col3, zeros1, zeros2)


def kernel(h, coords, edge_index, edge_attr, emb_in_W, emb_in_b,
           edge_W1, edge_b1, edge_W2, edge_b2,
           node_W1, node_b1, node_W2, node_b2,
           coord_W1, coord_b1, coord_W2, emb_out_W, emb_out_b):
    row3 = edge_index[0].reshape(NW, K, C)
    col3 = edge_index[1].reshape(NW, K, C)
    zeros1 = jnp.zeros((NP, HID), jnp.float32)
    zeros2 = jnp.zeros((NP, CP), jnp.float32)
    cpad = jnp.pad(coords, ((0, 0), (0, CP - 3)))
    cneg = -cpad

    x = _linear(h, emb_in_W.T, emb_in_b.reshape(1, HID))
    for l in range(4):
        eW1 = edge_W1[l]
        pf, qf = _prep(x, eW1[:, :HID].T, eW1[:, HID:2 * HID].T,
                       edge_b1[l].reshape(1, HID))
        gf, gc = _sc_gather(pf, qf, cneg, cpad, col3, row3)
        m1, m2 = _edge(gf, gc, edge_attr,
                       eW1[:, 2 * HID].reshape(1, HID),
                       eW1[:, 2 * HID + 1:].T,
                       edge_W2[l].T, edge_b2[l].reshape(1, HID),
                       coord_W1[l].T, coord_b1[l].reshape(1, HID),
                       coord_W2[l].reshape(1, HID))
        s1, s2 = _sc_scatter(m1, m2, col3, zeros1, zeros2)
        x, cpad, cneg = _node(x, cpad, s1, s2,
                              node_W1[l][:, :HID].T, node_W1[l][:, HID:].T,
                              node_b1[l].reshape(1, HID),
                              node_W2[l].T, node_b2[l].reshape(1, HID))
    x = _linear(x, emb_out_W.T, emb_out_b.reshape(1, HID))
    return (x, cpad[:, :3])


# speedup vs baseline: 1.2915x; 1.2915x over previous
import functools

import jax
import jax.numpy as jnp
from jax import lax
from jax.experimental import pallas as pl
from jax.experimental.pallas import tpu as pltpu
from jax.experimental.pallas import tpu_sc as plsc

N = 10000
E = 320000
HID = 128
CP = 16
NC = 2
NS = 16
NW = NC * NS
EW = E // NW
C = 80
K = EW // C
NP = 10112
TPR = NP // NS
BN = 2000
BE = 1280


def _silu(x):
    return x * jax.nn.sigmoid(x)


def _linear_body(x_ref, wt_ref, b_ref, o_ref):
    o_ref[...] = jnp.dot(x_ref[...], wt_ref[...],
                         preferred_element_type=jnp.float32) + b_ref[...]


def _linear(x, wt, b):
    return pl.pallas_call(
        _linear_body,
        grid=(N // BN,),
        in_specs=[pl.BlockSpec((BN, x.shape[1]), lambda i: (i, 0)),
                  pl.BlockSpec(wt.shape, lambda i: (0, 0)),
                  pl.BlockSpec((1, wt.shape[1]), lambda i: (0, 0))],
        out_specs=pl.BlockSpec((BN, wt.shape[1]), lambda i: (i, 0)),
        out_shape=jax.ShapeDtypeStruct((N, wt.shape[1]), jnp.float32),
    )(x, wt, b)


def _prep_body(x_ref, wct_ref, wrt_ref, b1_ref, p_ref, q_ref):
    x = x_ref[...]
    p_ref[...] = jnp.dot(x, wct_ref[...],
                         preferred_element_type=jnp.float32) + b1_ref[...]
    q_ref[...] = jnp.dot(x, wrt_ref[...], preferred_element_type=jnp.float32)


def _prep(x, wct, wrt, b1):
    return pl.pallas_call(
        _prep_body,
        grid=(N // BN,),
        in_specs=[pl.BlockSpec((BN, HID), lambda i: (i, 0)),
                  pl.BlockSpec((HID, HID), lambda i: (0, 0)),
                  pl.BlockSpec((HID, HID), lambda i: (0, 0)),
                  pl.BlockSpec((1, HID), lambda i: (0, 0))],
        out_specs=[pl.BlockSpec((BN, HID), lambda i: (i, 0)),
                   pl.BlockSpec((BN, HID), lambda i: (i, 0))],
        out_shape=[jax.ShapeDtypeStruct((N, HID), jnp.float32),
                   jax.ShapeDtypeStruct((N, HID), jnp.float32)],
    )(x, wct, wrt, b1)


def _edge_body(g1f_ref, g2f_ref, g1c_ref, g2c_ref, ea_ref, wd_ref, wat_ref,
               ew2t_ref, eb2_ref, cw1t_ref, cb1_ref, cw2_ref, m1_ref, m2_ref):
    cd = g1c_ref[...] + g2c_ref[...]
    cd3 = cd[:, :3]
    dist = jnp.sum(cd3 * cd3, axis=1, keepdims=True)
    pre = (g1f_ref[...] + g2f_ref[...] + dist * wd_ref[...]
           + jnp.dot(ea_ref[...], wat_ref[...],
                     preferred_element_type=jnp.float32))
    u = _silu(pre)
    ef = _silu(jnp.dot(u, ew2t_ref[...],
                       preferred_element_type=jnp.float32) + eb2_ref[...])
    t = _silu(jnp.dot(ef, cw1t_ref[...],
                      preferred_element_type=jnp.float32) + cb1_ref[...])
    cw = jnp.sum(t * cw2_ref[...], axis=1, keepdims=True)
    ones = jnp.ones((BE, 1), jnp.float32)
    zeros = jnp.zeros((BE, CP - 4), jnp.float32)
    m1_ref[...] = ef
    m2_ref[...] = jnp.concatenate([cd3 * cw, ones, zeros], axis=1)


def _edge(g1f, g2f, g1c, g2c, ea, wd, wat, ew2t, eb2, cw1t, cb1, cw2):
    return pl.pallas_call(
        _edge_body,
        grid=(E // BE,),
        in_specs=[pl.BlockSpec((BE, HID), lambda i: (i, 0)),
                  pl.BlockSpec((BE, HID), lambda i: (i, 0)),
                  pl.BlockSpec((BE, CP), lambda i: (i, 0)),
                  pl.BlockSpec((BE, CP), lambda i: (i, 0)),
                  pl.BlockSpec((BE, 4), lambda i: (i, 0)),
                  pl.BlockSpec((1, HID), lambda i: (0, 0)),
                  pl.BlockSpec((4, HID), lambda i: (0, 0)),
                  pl.BlockSpec((HID, HID), lambda i: (0, 0)),
                  pl.BlockSpec((1, HID), lambda i: (0, 0)),
                  pl.BlockSpec((HID, HID), lambda i: (0, 0)),
                  pl.BlockSpec((1, HID), lambda i: (0, 0)),
                  pl.BlockSpec((1, HID), lambda i: (0, 0))],
        out_specs=[pl.BlockSpec((BE, HID), lambda i: (i, 0)),
                   pl.BlockSpec((BE, CP), lambda i: (i, 0))],
        out_shape=[jax.ShapeDtypeStruct((E, HID), jnp.float32),
                   jax.ShapeDtypeStruct((E, CP), jnp.float32)],
    )(g1f, g2f, g1c, g2c, ea, wd, wat, ew2t, eb2, cw1t, cb1, cw2)


def _node_body(x_ref, cp_ref, s1a_ref, s1b_ref, s2a_ref, s2b_ref,
               w1at_ref, w1bt_ref, b1_ref, w2t_ref, b2_ref,
               xo_ref, cpo_ref, cno_ref):
    x = x_ref[...]
    agg = s1a_ref[0] + s1b_ref[0]
    t2 = s2a_ref[0] + s2b_ref[0]
    csum = t2[:, :3]
    cnt = t2[:, 3:4]
    aggc = csum / jnp.maximum(cnt, 1.0)
    pre = (jnp.dot(x, w1at_ref[...], preferred_element_type=jnp.float32)
           + jnp.dot(agg, w1bt_ref[...], preferred_element_type=jnp.float32)
           + b1_ref[...])
    upd = jnp.dot(_silu(pre), w2t_ref[...],
                  preferred_element_type=jnp.float32) + b2_ref[...]
    xo_ref[...] = x + upd
    cpo = cp_ref[...] + jnp.concatenate(
        [aggc, jnp.zeros((BN, CP - 3), jnp.float32)], axis=1)
    cpo_ref[...] = cpo
    cno_ref[...] = -cpo


def _node(x, cpad, s1, s2, w1at, w1bt, b1, w2t, b2):
    return pl.pallas_call(
        _node_body,
        grid=(N // BN,),
        in_specs=[pl.BlockSpec((BN, HID), lambda i: (i, 0)),
                  pl.BlockSpec((BN, CP), lambda i: (i, 0)),
                  pl.BlockSpec((1, BN, HID), lambda i: (0, i, 0)),
                  pl.BlockSpec((1, BN, HID), lambda i: (1, i, 0)),
                  pl.BlockSpec((1, BN, CP), lambda i: (0, i, 0)),
                  pl.BlockSpec((1, BN, CP), lambda i: (1, i, 0)),
                  pl.BlockSpec((HID, HID), lambda i: (0, 0)),
                  pl.BlockSpec((HID, HID), lambda i: (0, 0)),
                  pl.BlockSpec((1, HID), lambda i: (0, 0)),
                  pl.BlockSpec((HID, HID), lambda i: (0, 0)),
                  pl.BlockSpec((1, HID), lambda i: (0, 0))],
        out_specs=[pl.BlockSpec((BN, HID), lambda i: (i, 0)),
                   pl.BlockSpec((BN, CP), lambda i: (i, 0)),
                   pl.BlockSpec((BN, CP), lambda i: (i, 0))],
        out_shape=[jax.ShapeDtypeStruct((N, HID), jnp.float32),
                   jax.ShapeDtypeStruct((N, CP), jnp.float32),
                   jax.ShapeDtypeStruct((N, CP), jnp.float32)],
    )(x, cpad, s1, s1, s2, s2, w1at, w1bt, b1, w2t, b2)


@functools.lru_cache(maxsize=1)
def _sc_mesh():
    return plsc.VectorSubcoreMesh(core_axis_name="c", subcore_axis_name="s")


def _gather_body(pf_hbm, qf_hbm, pc_hbm, qc_hbm, col_hbm, row_hbm,
                 g1f_hbm, g2f_hbm, g1c_hbm, g2c_hbm,
                 idxc, idxr,
                 buf_af0, buf_bf0, buf_ac0, buf_bc0,
                 buf_af1, buf_bf1, buf_ac1, buf_bc1,
                 sem0, sem1):
    wid = lax.axis_index("s") * NC + lax.axis_index("c")
    pltpu.sync_copy(col_hbm.at[wid], idxc)
    pltpu.sync_copy(row_hbm.at[wid], idxr)

    def fire(j, bf, bff, bc, bcc, sem):
        pltpu.async_copy(pf_hbm.at[idxc.at[j]], bf, sem)
        pltpu.async_copy(qf_hbm.at[idxr.at[j]], bff, sem)
        pltpu.async_copy(pc_hbm.at[idxc.at[j]], bc, sem)
        pltpu.async_copy(qc_hbm.at[idxr.at[j]], bcc, sem)

    def wait_bank(bf, bff, bc, bcc, sem):
        pltpu.make_async_copy(pf_hbm.at[pl.ds(0, C)], bf, sem).wait()
        pltpu.make_async_copy(qf_hbm.at[pl.ds(0, C)], bff, sem).wait()
        pltpu.make_async_copy(pc_hbm.at[pl.ds(0, C)], bc, sem).wait()
        pltpu.make_async_copy(qc_hbm.at[pl.ds(0, C)], bcc, sem).wait()

    def write(j, bf, bc):
        base = wid * EW + j * C
        pltpu.sync_copy(bf, g1f_hbm.at[pl.ds(base, C)])
        pltpu.sync_copy(bc, g1c_hbm.at[pl.ds(base, C)])

    def write2(j, bff, bcc):
        base = wid * EW + j * C
        pltpu.sync_copy(bff, g2f_hbm.at[pl.ds(base, C)])
        pltpu.sync_copy(bcc, g2c_hbm.at[pl.ds(base, C)])

    fire(0, buf_af0, buf_bf0, buf_ac0, buf_bc0, sem0)

    def body(jj, carry):
        j0 = 2 * jj
        j1 = 2 * jj + 1
        fire(j1, buf_af1, buf_bf1, buf_ac1, buf_bc1, sem1)
        wait_bank(buf_af0, buf_bf0, buf_ac0, buf_bc0, sem0)
        write(j0, buf_af0, buf_ac0)
        write2(j0, buf_bf0, buf_bc0)
        fire(j0 + 2, buf_af0, buf_bf0, buf_ac0, buf_bc0, sem0)
        wait_bank(buf_af1, buf_bf1, buf_ac1, buf_bc1, sem1)
        write(j1, buf_af1, buf_ac1)
        write2(j1, buf_bf1, buf_bc1)
        return carry

    lax.fori_loop(0, K // 2, body, 0)
    wait_bank(buf_af0, buf_bf0, buf_ac0, buf_bc0, sem0)
    write(K - 1, buf_af0, buf_ac0)
    write2(K - 1, buf_bf0, buf_bc0)


def _sc_gather(pf, qf, pc, qc, col3, row3):
    kfn = pl.kernel(
        _gather_body,
        out_type=[jax.ShapeDtypeStruct((E, HID), jnp.float32),
                  jax.ShapeDtypeStruct((E, HID), jnp.float32),
                  jax.ShapeDtypeStruct((E, CP), jnp.float32),
                  jax.ShapeDtypeStruct((E, CP), jnp.float32)],
        mesh=_sc_mesh(),
        scratch_types=[pltpu.VMEM((K, C), jnp.int32),
                       pltpu.VMEM((K, C), jnp.int32),
                       pltpu.VMEM((C, HID), jnp.float32),
                       pltpu.VMEM((C, HID), jnp.float32),
                       pltpu.VMEM((C, CP), jnp.float32),
                       pltpu.VMEM((C, CP), jnp.float32),
                       pltpu.VMEM((C, HID), jnp.float32),
                       pltpu.VMEM((C, HID), jnp.float32),
                       pltpu.VMEM((C, CP), jnp.float32),
                       pltpu.VMEM((C, CP), jnp.float32),
                       pltpu.SemaphoreType.DMA,
                       pltpu.SemaphoreType.DMA],
        compiler_params=pltpu.CompilerParams(use_tc_tiling_on_sc=False),
    )
    return kfn(pf, qf, pc, qc, col3, row3)


def _scatter_body(m1_hbm, m2_hbm, col_hbm, z1_hbm, z2_hbm, s1_hbm, s2_hbm,
                  idxc, buf10, buf20, buf11, buf21, acc1, acc2, sem0, sem1):
    cid = lax.axis_index("c")
    sid = lax.axis_index("s")
    wid = sid * NC + cid
    pltpu.sync_copy(z1_hbm.at[pl.ds(sid * TPR, TPR)],
                    acc1.at[pl.ds(sid * TPR, TPR)])
    pltpu.sync_copy(z2_hbm.at[pl.ds(sid * TPR, TPR)],
                    acc2.at[pl.ds(sid * TPR, TPR)])
    pltpu.sync_copy(col_hbm.at[wid], idxc)
    plsc.subcore_barrier()

    def fire(j, b1, b2, sem):
        base = wid * EW + j * C
        pltpu.async_copy(m1_hbm.at[pl.ds(base, C)], b1, sem)
        pltpu.async_copy(m2_hbm.at[pl.ds(base, C)], b2, sem)

    def wait_bank(b1, b2, sem):
        pltpu.make_async_copy(m1_hbm.at[pl.ds(0, C)], b1, sem).wait()
        pltpu.make_async_copy(m2_hbm.at[pl.ds(0, C)], b2, sem).wait()

    def scat(j, b1, b2):
        pltpu.sync_copy(b1, acc1.at[idxc.at[j]], add=True)
        pltpu.sync_copy(b2, acc2.at[idxc.at[j]], add=True)

    fire(0, buf10, buf20, sem0)

    def body(jj, carry):
        j0 = 2 * jj
        j1 = 2 * jj + 1
        fire(j1, buf11, buf21, sem1)
        wait_bank(buf10, buf20, sem0)
        scat(j0, buf10, buf20)
        fire(j0 + 2, buf10, buf20, sem0)
        wait_bank(buf11, buf21, sem1)
        scat(j1, buf11, buf21)
        return carry

    lax.fori_loop(0, K // 2, body, 0)
    wait_bank(buf10, buf20, sem0)
    scat(K - 1, buf10, buf20)
    plsc.subcore_barrier()
    pltpu.sync_copy(acc1.at[pl.ds(sid * TPR, TPR)],
                    s1_hbm.at[cid, pl.ds(sid * TPR, TPR)])
    pltpu.sync_copy(acc2.at[pl.ds(sid * TPR, TPR)],
                    s2_hbm.at[cid, pl.ds(sid * TPR, TPR)])


def _sc_scatter(m1, m2, col3, zeros1, zeros2):
    kfn = pl.kernel(
        _scatter_body,
        out_type=[jax.ShapeDtypeStruct((2, NP, HID), jnp.float32),
                  jax.ShapeDtypeStruct((2, NP, CP), jnp.float32)],
        mesh=_sc_mesh(),
        scratch_types=[pltpu.VMEM((K, C), jnp.int32),
                       pltpu.VMEM((C, HID), jnp.float32),
                       pltpu.VMEM((C, CP), jnp.float32),
                       pltpu.VMEM((C, HID), jnp.float32),
                       pltpu.VMEM((C, CP), jnp.float32),
                       pltpu.VMEM_SHARED((NP, HID), jnp.float32),
                       pltpu.VMEM_SHARED((NP, CP), jnp.float32),
                       pltpu.SemaphoreType.DMA,
                       pltpu.SemaphoreType.DMA],
        compiler_params=pltpu.CompilerParams(use_tc_tiling_on_sc=False),
    )
    return kfn(m1, m2, col3, zeros1, zeros2)


def kernel(h, coords, edge_index, edge_attr, emb_in_W, emb_in_b,
           edge_W1, edge_b1, edge_W2, edge_b2,
           node_W1, node_b1, node_W2, node_b2,
           coord_W1, coord_b1, coord_W2, emb_out_W, emb_out_b):
    row3 = edge_index[0].reshape(NW, K, C)
    col3 = edge_index[1].reshape(NW, K, C)
    zeros1 = jnp.zeros((NP, HID), jnp.float32)
    zeros2 = jnp.zeros((NP, CP), jnp.float32)
    cpad = jnp.pad(coords, ((0, 0), (0, CP - 3)))
    cneg = -cpad

    x = _linear(h, emb_in_W.T, emb_in_b.reshape(1, HID))
    for l in range(4):
        eW1 = edge_W1[l]
        pf, qf = _prep(x, eW1[:, :HID].T, eW1[:, HID:2 * HID].T,
                       edge_b1[l].reshape(1, HID))
        g1f, g2f, g1c, g2c = _sc_gather(pf, qf, cneg, cpad, col3, row3)
        m1, m2 = _edge(g1f, g2f, g1c, g2c, edge_attr,
                       eW1[:, 2 * HID].reshape(1, HID),
                       eW1[:, 2 * HID + 1:].T,
                       edge_W2[l].T, edge_b2[l].reshape(1, HID),
                       coord_W1[l].T, coord_b1[l].reshape(1, HID),
                       coord_W2[l].reshape(1, HID))
        s1, s2 = _sc_scatter(m1, m2, col3, zeros1, zeros2)
        x, cpad, cneg = _node(x, cpad, s1, s2,
                              node_W1[l][:, :HID].T, node_W1[l][:, HID:].T,
                              node_b1[l].reshape(1, HID),
                              node_W2[l].T, node_b2[l].reshape(1, HID))
    x = _linear(x, emb_out_W.T, emb_out_b.reshape(1, HID))
    return (x, cpad[:, :3])


# speedup vs baseline: 1.3189x; 1.0212x over previous
import functools

import jax
import jax.numpy as jnp
from jax import lax
from jax.experimental import pallas as pl
from jax.experimental.pallas import tpu as pltpu
from jax.experimental.pallas import tpu_sc as plsc

N = 10000
E = 320000
HID = 128
CP = 16
NC = 2
NS = 16
NW = NC * NS
EW = E // NW
C = 80
K = EW // C
NP = 10112
TPR = NP // NS
BN = 2000
BE = 1280


def _silu(x):
    return x * jax.nn.sigmoid(x)


def _linear_body(x_ref, wt_ref, b_ref, o_ref):
    o_ref[...] = jnp.dot(x_ref[...], wt_ref[...],
                         preferred_element_type=jnp.float32) + b_ref[...]


def _linear(x, wt, b):
    return pl.pallas_call(
        _linear_body,
        grid=(N // BN,),
        in_specs=[pl.BlockSpec((BN, x.shape[1]), lambda i: (i, 0)),
                  pl.BlockSpec(wt.shape, lambda i: (0, 0)),
                  pl.BlockSpec((1, wt.shape[1]), lambda i: (0, 0))],
        out_specs=pl.BlockSpec((BN, wt.shape[1]), lambda i: (i, 0)),
        out_shape=jax.ShapeDtypeStruct((N, wt.shape[1]), jnp.float32),
    )(x, wt, b)


def _prep_body(x_ref, wct_ref, wrt_ref, b1_ref, p_ref, q_ref):
    x = x_ref[...]
    p_ref[...] = jnp.dot(x, wct_ref[...],
                         preferred_element_type=jnp.float32) + b1_ref[...]
    q_ref[...] = jnp.dot(x, wrt_ref[...], preferred_element_type=jnp.float32)


def _prep(x, wct, wrt, b1):
    return pl.pallas_call(
        _prep_body,
        grid=(N // BN,),
        in_specs=[pl.BlockSpec((BN, HID), lambda i: (i, 0)),
                  pl.BlockSpec((HID, HID), lambda i: (0, 0)),
                  pl.BlockSpec((HID, HID), lambda i: (0, 0)),
                  pl.BlockSpec((1, HID), lambda i: (0, 0))],
        out_specs=[pl.BlockSpec((BN, HID), lambda i: (i, 0)),
                   pl.BlockSpec((BN, HID), lambda i: (i, 0))],
        out_shape=[jax.ShapeDtypeStruct((N, HID), jnp.float32),
                   jax.ShapeDtypeStruct((N, HID), jnp.float32)],
    )(x, wct, wrt, b1)


def _edge_body(g1f_ref, g2f_ref, g1c_ref, g2c_ref, ea_ref, wd_ref, wat_ref,
               ew2t_ref, eb2_ref, cw1t_ref, cb1_ref, cw2_ref, m1_ref, m2_ref):
    cd = g1c_ref[...] + g2c_ref[...]
    cd3 = cd[:, :3]
    dist = jnp.sum(cd3 * cd3, axis=1, keepdims=True)
    pre = (g1f_ref[...] + g2f_ref[...] + dist * wd_ref[...]
           + jnp.dot(ea_ref[...], wat_ref[...],
                     preferred_element_type=jnp.float32))
    u = _silu(pre)
    ef = _silu(jnp.dot(u, ew2t_ref[...],
                       preferred_element_type=jnp.float32) + eb2_ref[...])
    t = _silu(jnp.dot(ef, cw1t_ref[...],
                      preferred_element_type=jnp.float32) + cb1_ref[...])
    cw = jnp.sum(t * cw2_ref[...], axis=1, keepdims=True)
    ones = jnp.ones((BE, 1), jnp.float32)
    zeros = jnp.zeros((BE, CP - 4), jnp.float32)
    m1_ref[...] = ef
    m2_ref[...] = jnp.concatenate([cd3 * cw, ones, zeros], axis=1)


def _edge(g1f, g2f, g1c, g2c, ea, wd, wat, ew2t, eb2, cw1t, cb1, cw2):
    ne = g1f.shape[0]
    return pl.pallas_call(
        _edge_body,
        grid=(ne // BE,),
        in_specs=[pl.BlockSpec((BE, HID), lambda i: (i, 0)),
                  pl.BlockSpec((BE, HID), lambda i: (i, 0)),
                  pl.BlockSpec((BE, CP), lambda i: (i, 0)),
                  pl.BlockSpec((BE, CP), lambda i: (i, 0)),
                  pl.BlockSpec((BE, 4), lambda i: (i, 0)),
                  pl.BlockSpec((1, HID), lambda i: (0, 0)),
                  pl.BlockSpec((4, HID), lambda i: (0, 0)),
                  pl.BlockSpec((HID, HID), lambda i: (0, 0)),
                  pl.BlockSpec((1, HID), lambda i: (0, 0)),
                  pl.BlockSpec((HID, HID), lambda i: (0, 0)),
                  pl.BlockSpec((1, HID), lambda i: (0, 0)),
                  pl.BlockSpec((1, HID), lambda i: (0, 0))],
        out_specs=[pl.BlockSpec((BE, HID), lambda i: (i, 0)),
                   pl.BlockSpec((BE, CP), lambda i: (i, 0))],
        out_shape=[jax.ShapeDtypeStruct((ne, HID), jnp.float32),
                   jax.ShapeDtypeStruct((ne, CP), jnp.float32)],
    )(g1f, g2f, g1c, g2c, ea, wd, wat, ew2t, eb2, cw1t, cb1, cw2)


def _node_body(x_ref, cp_ref, s1a_ref, s1b_ref, s1c_ref, s1d_ref,
               s2a_ref, s2b_ref, s2c_ref, s2d_ref,
               w1at_ref, w1bt_ref, b1_ref, w2t_ref, b2_ref,
               xo_ref, cpo_ref, cno_ref):
    x = x_ref[...]
    agg = (s1a_ref[0] + s1b_ref[0]) + (s1c_ref[0] + s1d_ref[0])
    t2 = (s2a_ref[0] + s2b_ref[0]) + (s2c_ref[0] + s2d_ref[0])
    csum = t2[:, :3]
    cnt = t2[:, 3:4]
    aggc = csum / jnp.maximum(cnt, 1.0)
    pre = (jnp.dot(x, w1at_ref[...], preferred_element_type=jnp.float32)
           + jnp.dot(agg, w1bt_ref[...], preferred_element_type=jnp.float32)
           + b1_ref[...])
    upd = jnp.dot(_silu(pre), w2t_ref[...],
                  preferred_element_type=jnp.float32) + b2_ref[...]
    xo_ref[...] = x + upd
    cpo = cp_ref[...] + jnp.concatenate(
        [aggc, jnp.zeros((BN, CP - 3), jnp.float32)], axis=1)
    cpo_ref[...] = cpo
    cno_ref[...] = -cpo


def _node(x, cpad, s1a, s1b, s2a, s2b, w1at, w1bt, b1, w2t, b2):
    return pl.pallas_call(
        _node_body,
        grid=(N // BN,),
        in_specs=[pl.BlockSpec((BN, HID), lambda i: (i, 0)),
                  pl.BlockSpec((BN, CP), lambda i: (i, 0)),
                  pl.BlockSpec((1, BN, HID), lambda i: (0, i, 0)),
                  pl.BlockSpec((1, BN, HID), lambda i: (1, i, 0)),
                  pl.BlockSpec((1, BN, HID), lambda i: (0, i, 0)),
                  pl.BlockSpec((1, BN, HID), lambda i: (1, i, 0)),
                  pl.BlockSpec((1, BN, CP), lambda i: (0, i, 0)),
                  pl.BlockSpec((1, BN, CP), lambda i: (1, i, 0)),
                  pl.BlockSpec((1, BN, CP), lambda i: (0, i, 0)),
                  pl.BlockSpec((1, BN, CP), lambda i: (1, i, 0)),
                  pl.BlockSpec((HID, HID), lambda i: (0, 0)),
                  pl.BlockSpec((HID, HID), lambda i: (0, 0)),
                  pl.BlockSpec((1, HID), lambda i: (0, 0)),
                  pl.BlockSpec((HID, HID), lambda i: (0, 0)),
                  pl.BlockSpec((1, HID), lambda i: (0, 0))],
        out_specs=[pl.BlockSpec((BN, HID), lambda i: (i, 0)),
                   pl.BlockSpec((BN, CP), lambda i: (i, 0)),
                   pl.BlockSpec((BN, CP), lambda i: (i, 0))],
        out_shape=[jax.ShapeDtypeStruct((N, HID), jnp.float32),
                   jax.ShapeDtypeStruct((N, CP), jnp.float32),
                   jax.ShapeDtypeStruct((N, CP), jnp.float32)],
    )(x, cpad, s1a, s1a, s1b, s1b, s2a, s2a, s2b, s2b,
      w1at, w1bt, b1, w2t, b2)


@functools.lru_cache(maxsize=1)
def _sc_mesh():
    return plsc.VectorSubcoreMesh(core_axis_name="c", subcore_axis_name="s")


def _sc_gather(pf, qf, pc, qc, col3, row3):
    k, c = col3.shape[1], col3.shape[2]
    ew = k * c
    ne = NW * ew

    def body(pf_hbm, qf_hbm, pc_hbm, qc_hbm, col_hbm, row_hbm,
             g1f_hbm, g2f_hbm, g1c_hbm, g2c_hbm,
             idxc, idxr,
             buf_af0, buf_bf0, buf_ac0, buf_bc0,
             buf_af1, buf_bf1, buf_ac1, buf_bc1,
             sem0, sem1):
        wid = lax.axis_index("s") * NC + lax.axis_index("c")
        pltpu.sync_copy(col_hbm.at[wid], idxc)
        pltpu.sync_copy(row_hbm.at[wid], idxr)

        def fire(j, bf, bff, bc, bcc, sem):
            pltpu.async_copy(pf_hbm.at[idxc.at[j]], bf, sem)
            pltpu.async_copy(qf_hbm.at[idxr.at[j]], bff, sem)
            pltpu.async_copy(pc_hbm.at[idxc.at[j]], bc, sem)
            pltpu.async_copy(qc_hbm.at[idxr.at[j]], bcc, sem)

        def wait_bank(bf, bff, bc, bcc, sem):
            pltpu.make_async_copy(pf_hbm.at[pl.ds(0, c)], bf, sem).wait()
            pltpu.make_async_copy(qf_hbm.at[pl.ds(0, c)], bff, sem).wait()
            pltpu.make_async_copy(pc_hbm.at[pl.ds(0, c)], bc, sem).wait()
            pltpu.make_async_copy(qc_hbm.at[pl.ds(0, c)], bcc, sem).wait()

        def write(j, bf, bff, bc, bcc):
            base = wid * ew + j * c
            pltpu.sync_copy(bf, g1f_hbm.at[pl.ds(base, c)])
            pltpu.sync_copy(bc, g1c_hbm.at[pl.ds(base, c)])
            pltpu.sync_copy(bff, g2f_hbm.at[pl.ds(base, c)])
            pltpu.sync_copy(bcc, g2c_hbm.at[pl.ds(base, c)])

        fire(0, buf_af0, buf_bf0, buf_ac0, buf_bc0, sem0)

        def loop(jj, carry):
            j0 = 2 * jj
            j1 = 2 * jj + 1
            fire(j1, buf_af1, buf_bf1, buf_ac1, buf_bc1, sem1)
            wait_bank(buf_af0, buf_bf0, buf_ac0, buf_bc0, sem0)
            write(j0, buf_af0, buf_bf0, buf_ac0, buf_bc0)
            fire(j0 + 2, buf_af0, buf_bf0, buf_ac0, buf_bc0, sem0)
            wait_bank(buf_af1, buf_bf1, buf_ac1, buf_bc1, sem1)
            write(j1, buf_af1, buf_bf1, buf_ac1, buf_bc1)
            return carry

        lax.fori_loop(0, k // 2, loop, 0)
        wait_bank(buf_af0, buf_bf0, buf_ac0, buf_bc0, sem0)
        write(k - 1, buf_af0, buf_bf0, buf_ac0, buf_bc0)

    kfn = pl.kernel(
        body,
        out_type=[jax.ShapeDtypeStruct((ne, HID), jnp.float32),
                  jax.ShapeDtypeStruct((ne, HID), jnp.float32),
                  jax.ShapeDtypeStruct((ne, CP), jnp.float32),
                  jax.ShapeDtypeStruct((ne, CP), jnp.float32)],
        mesh=_sc_mesh(),
        scratch_types=[pltpu.VMEM((k, c), jnp.int32),
                       pltpu.VMEM((k, c), jnp.int32),
                       pltpu.VMEM((c, HID), jnp.float32),
                       pltpu.VMEM((c, HID), jnp.float32),
                       pltpu.VMEM((c, CP), jnp.float32),
                       pltpu.VMEM((c, CP), jnp.float32),
                       pltpu.VMEM((c, HID), jnp.float32),
                       pltpu.VMEM((c, HID), jnp.float32),
                       pltpu.VMEM((c, CP), jnp.float32),
                       pltpu.VMEM((c, CP), jnp.float32),
                       pltpu.SemaphoreType.DMA,
                       pltpu.SemaphoreType.DMA],
        compiler_params=pltpu.CompilerParams(use_tc_tiling_on_sc=False),
    )
    return kfn(pf, qf, pc, qc, col3, row3)


def _sc_scatter(m1, m2, col3, zeros1, zeros2):
    k, c = col3.shape[1], col3.shape[2]
    ew = k * c

    def body(m1_hbm, m2_hbm, col_hbm, z1_hbm, z2_hbm, s1_hbm, s2_hbm,
             idxc, buf10, buf20, buf11, buf21, acc1, acc2, sem0, sem1):
        cid = lax.axis_index("c")
        sid = lax.axis_index("s")
        wid = sid * NC + cid
        pltpu.sync_copy(z1_hbm.at[pl.ds(sid * TPR, TPR)],
                        acc1.at[pl.ds(sid * TPR, TPR)])
        pltpu.sync_copy(z2_hbm.at[pl.ds(sid * TPR, TPR)],
                        acc2.at[pl.ds(sid * TPR, TPR)])
        pltpu.sync_copy(col_hbm.at[wid], idxc)
        plsc.subcore_barrier()

        def fire(j, b1, b2, sem):
            base = wid * ew + j * c
            pltpu.async_copy(m1_hbm.at[pl.ds(base, c)], b1, sem)
            pltpu.async_copy(m2_hbm.at[pl.ds(base, c)], b2, sem)

        def wait_bank(b1, b2, sem):
            pltpu.make_async_copy(m1_hbm.at[pl.ds(0, c)], b1, sem).wait()
            pltpu.make_async_copy(m2_hbm.at[pl.ds(0, c)], b2, sem).wait()

        def scat(j, b1, b2):
            pltpu.sync_copy(b1, acc1.at[idxc.at[j]], add=True)
            pltpu.sync_copy(b2, acc2.at[idxc.at[j]], add=True)

        fire(0, buf10, buf20, sem0)

        def loop(jj, carry):
            j0 = 2 * jj
            j1 = 2 * jj + 1
            fire(j1, buf11, buf21, sem1)
            wait_bank(buf10, buf20, sem0)
            scat(j0, buf10, buf20)
            fire(j0 + 2, buf10, buf20, sem0)
            wait_bank(buf11, buf21, sem1)
            scat(j1, buf11, buf21)
            return carry

        lax.fori_loop(0, k // 2, loop, 0)
        wait_bank(buf10, buf20, sem0)
        scat(k - 1, buf10, buf20)
        plsc.subcore_barrier()
        pltpu.sync_copy(acc1.at[pl.ds(sid * TPR, TPR)],
                        s1_hbm.at[cid, pl.ds(sid * TPR, TPR)])
        pltpu.sync_copy(acc2.at[pl.ds(sid * TPR, TPR)],
                        s2_hbm.at[cid, pl.ds(sid * TPR, TPR)])

    kfn = pl.kernel(
        body,
        out_type=[jax.ShapeDtypeStruct((2, NP, HID), jnp.float32),
                  jax.ShapeDtypeStruct((2, NP, CP), jnp.float32)],
        mesh=_sc_mesh(),
        scratch_types=[pltpu.VMEM((k, c), jnp.int32),
                       pltpu.VMEM((c, HID), jnp.float32),
                       pltpu.VMEM((c, CP), jnp.float32),
                       pltpu.VMEM((c, HID), jnp.float32),
                       pltpu.VMEM((c, CP), jnp.float32),
                       pltpu.VMEM_SHARED((NP, HID), jnp.float32),
                       pltpu.VMEM_SHARED((NP, CP), jnp.float32),
                       pltpu.SemaphoreType.DMA,
                       pltpu.SemaphoreType.DMA],
        compiler_params=pltpu.CompilerParams(use_tc_tiling_on_sc=False),
    )
    return kfn(m1, m2, col3, zeros1, zeros2)


def kernel(h, coords, edge_index, edge_attr, emb_in_W, emb_in_b,
           edge_W1, edge_b1, edge_W2, edge_b2,
           node_W1, node_b1, node_W2, node_b2,
           coord_W1, coord_b1, coord_W2, emb_out_W, emb_out_b):
    eh = E // 2
    ch = 40
    kh = eh // NW // ch
    row4 = edge_index[0].reshape(2, NW, kh, ch)
    col4 = edge_index[1].reshape(2, NW, kh, ch)
    rows = (row4[0], row4[1])
    cols = (col4[0], col4[1])
    eas = (edge_attr[:eh], edge_attr[eh:])
    zeros1 = jnp.zeros((NP, HID), jnp.float32)
    zeros2 = jnp.zeros((NP, CP), jnp.float32)
    cpad = jnp.pad(coords, ((0, 0), (0, CP - 3)))
    cneg = -cpad

    x = _linear(h, emb_in_W.T, emb_in_b.reshape(1, HID))
    for l in range(4):
        eW1 = edge_W1[l]
        pf, qf = _prep(x, eW1[:, :HID].T, eW1[:, HID:2 * HID].T,
                       edge_b1[l].reshape(1, HID))
        ew = (eW1[:, 2 * HID].reshape(1, HID), eW1[:, 2 * HID + 1:].T,
              edge_W2[l].T, edge_b2[l].reshape(1, HID),
              coord_W1[l].T, coord_b1[l].reshape(1, HID),
              coord_W2[l].reshape(1, HID))
        g0 = _sc_gather(pf, qf, cneg, cpad, cols[0], rows[0])
        g1 = _sc_gather(pf, qf, cneg, cpad, cols[1], rows[1])
        m0 = _edge(*g0, eas[0], *ew)
        m1 = _edge(*g1, eas[1], *ew)
        s10, s20 = _sc_scatter(m0[0], m0[1], cols[0], zeros1, zeros2)
        s11, s21 = _sc_scatter(m1[0], m1[1], cols[1], zeros1, zeros2)
        x, cpad, cneg = _node(x, cpad, s10, s11, s20, s21,
                              node_W1[l][:, :HID].T, node_W1[l][:, HID:].T,
                              node_b1[l].reshape(1, HID),
                              node_W2[l].T, node_b2[l].reshape(1, HID))
    x = _linear(x, emb_out_W.T, emb_out_b.reshape(1, HID))
    return (x, cpad[:, :3])
